# baseline (speedup 1.0000x reference)
import jax
import jax.numpy as jnp
from jax.experimental import pallas as pl

B = 16384
D = 64
BR = 2048


def _body(gu_ref, gi_ref, bu_ref, bi_ref, mu_ref, out_ref):
    s = jnp.sum(gu_ref[...] * gi_ref[...], axis=1, keepdims=True)
    out_ref[...] = s + bu_ref[...] + bi_ref[...] + mu_ref[...]


def kernel(gu, gi, bu, bi, Mu):
    out = pl.pallas_call(
        _body,
        grid=(B // BR,),
        in_specs=[
            pl.BlockSpec((BR, D), lambda i: (i, 0)),
            pl.BlockSpec((BR, D), lambda i: (i, 0)),
            pl.BlockSpec((BR, 1), lambda i: (i, 0)),
            pl.BlockSpec((BR, 1), lambda i: (i, 0)),
            pl.BlockSpec((1, 1), lambda i: (0, 0)),
        ],
        out_specs=pl.BlockSpec((BR, 1), lambda i: (i, 0)),
        out_shape=jax.ShapeDtypeStruct((B, 1), jnp.float32),
    )(gu, gi, bu, bi, Mu)
    return out[:, 0]


# speedup vs baseline: 1.4827x; 1.4827x over previous
import jax
import jax.numpy as jnp
from jax.experimental import pallas as pl

B = 16384
D = 64
BR = 2048


def _body(gu_ref, gi_ref, bu_ref, bi_ref, mu_ref, out_ref):
    s = jnp.sum(gu_ref[...] * gi_ref[...], axis=1)
    out_ref[...] = s + bu_ref[...] + bi_ref[...] + mu_ref[0]


def kernel(gu, gi, bu, bi, Mu):
    bu1 = bu.reshape(B)
    bi1 = bi.reshape(B)
    mu1 = Mu.reshape(1)
    out = pl.pallas_call(
        _body,
        grid=(B // BR,),
        in_specs=[
            pl.BlockSpec((BR, D), lambda i: (i, 0)),
            pl.BlockSpec((BR, D), lambda i: (i, 0)),
            pl.BlockSpec((BR,), lambda i: (i,)),
            pl.BlockSpec((BR,), lambda i: (i,)),
            pl.BlockSpec((1,), lambda i: (0,)),
        ],
        out_specs=pl.BlockSpec((BR,), lambda i: (i,)),
        out_shape=jax.ShapeDtypeStruct((B,), jnp.float32),
    )(gu, gi, bu1, bi1, mu1)
    return out
